# Initial kernel scaffold; baseline (speedup 1.0000x reference)
import functools

import jax
import jax.numpy as jnp
from jax import lax
from jax.experimental import pallas as pl
from jax.experimental.pallas import tpu as pltpu
from jax.experimental.pallas import tpu_sc as plsc

NC = 2
NS = 16
NW = NC * NS


def _sc_mesh():
    return plsc.VectorSubcoreMesh(core_axis_name="c", subcore_axis_name="s")


def _hist_kernel(n_pad: int, e: int, ch: int):
    per_tile = e // NW
    nch = per_tile // ch
    slc = n_pad // NS

    @functools.partial(
        pl.kernel,
        out_type=jax.ShapeDtypeStruct((NC, n_pad), jnp.float32),
        mesh=_sc_mesh(),
        scratch_types=[
            pltpu.VMEM((ch,), jnp.int32),
            pltpu.VMEM((ch,), jnp.float32),
            pltpu.VMEM_SHARED((n_pad,), jnp.float32),
        ],
    )
    def hist(ei_hbm, z_hbm, out_hbm, idx_v, ones_v, deg_sh):
        c = lax.axis_index("c")
        s = lax.axis_index("s")

        @pl.loop(0, ch, step=16)
        def _(i):
            ones_v[pl.ds(i, 16)] = jnp.full((16,), 1.0, jnp.float32)

        pltpu.sync_copy(z_hbm.at[pl.ds(s * slc, slc)],
                        deg_sh.at[pl.ds(s * slc, slc)])
        plsc.subcore_barrier()

        base = (c * NS + s) * per_tile

        @pl.loop(0, nch)
        def _(k):
            pltpu.sync_copy(ei_hbm.at[1, pl.ds(base + k * ch, ch)], idx_v)
            pltpu.sync_copy(ones_v, deg_sh.at[idx_v], add=True)

        plsc.subcore_barrier()
        pltpu.sync_copy(deg_sh.at[pl.ds(s * slc, slc)],
                        out_hbm.at[c, pl.ds(s * slc, slc)])

    return hist


def _msg_kernel(n_pad: int, e: int, ch: int):
    per_tile = e // NW
    nch = per_tile // ch
    slc = n_pad // NS

    @functools.partial(
        pl.kernel,
        out_type=jax.ShapeDtypeStruct((NC, n_pad, 2), jnp.float32),
        mesh=_sc_mesh(),
        scratch_types=[
            pltpu.VMEM((ch,), jnp.int32),
            pltpu.VMEM((ch,), jnp.int32),
            pltpu.VMEM((ch, 2), jnp.float32),
            pltpu.VMEM_SHARED((n_pad, 2), jnp.float32),
        ],
    )
    def msg(ei_hbm, q_hbm, z2_hbm, out_hbm, src_v, dst_v, msg_v, acc_sh):
        c = lax.axis_index("c")
        s = lax.axis_index("s")

        pltpu.sync_copy(z2_hbm.at[pl.ds(s * slc, slc)],
                        acc_sh.at[pl.ds(s * slc, slc)])
        plsc.subcore_barrier()

        base = (c * NS + s) * per_tile

        @pl.loop(0, nch)
        def _(k):
            pltpu.sync_copy(ei_hbm.at[0, pl.ds(base + k * ch, ch)], src_v)
            pltpu.sync_copy(ei_hbm.at[1, pl.ds(base + k * ch, ch)], dst_v)
            pltpu.sync_copy(q_hbm.at[src_v], msg_v)
            pltpu.sync_copy(msg_v, acc_sh.at[dst_v], add=True)

        plsc.subcore_barrier()
        pltpu.sync_copy(acc_sh.at[pl.ds(s * slc, slc)],
                        out_hbm.at[c, pl.ds(s * slc, slc)])

    return msg


def _glue_a(x2_ref, p_ref, dinv_ref, q_ref):
    deg = p_ref[0] + p_ref[1] + 1.0
    dinv = lax.rsqrt(deg)
    dinv_ref[...] = dinv
    q_ref[0] = x2_ref[0] * dinv
    q_ref[1] = x2_ref[1] * dinv


def _glue_b(acc_ref, q_ref, dinv_ref, w_ref, b_ref, o_ref):
    s0 = acc_ref[0, 0] + acc_ref[1, 0] + q_ref[0]
    s1 = acc_ref[0, 1] + acc_ref[1, 1] + q_ref[1]
    dinv = dinv_ref[...]
    for k in range(3):
        z = dinv * (s0 * w_ref[0, k] + s1 * w_ref[1, k]) + b_ref[0, k]
        o_ref[k] = jax.nn.sigmoid(z)


def kernel(x, edge_index, W, b):
    n = x.shape[0]
    e = edge_index.shape[1]
    n_pad = -(-n // 128) * 128
    while n_pad % (NS * 8) != 0:
        n_pad += 128
    r = n_pad // 128
    ch = 8000
    assert e % (NW * ch) == 0

    z1 = jnp.zeros((n_pad,), jnp.float32)
    p = _hist_kernel(n_pad, e, ch)(edge_index, z1)

    xt = jnp.zeros((2, n_pad), jnp.float32).at[:, :n].set(x.T)
    x2 = xt.reshape(2, r, 128)
    p2 = p.reshape(NC, r, 128)
    dinv, qf = pl.pallas_call(
        _glue_a,
        out_shape=[
            jax.ShapeDtypeStruct((r, 128), jnp.float32),
            jax.ShapeDtypeStruct((2, r, 128), jnp.float32),
        ],
    )(x2, p2)

    q_rows = qf.reshape(2, n_pad).T
    z2 = jnp.zeros((n_pad, 2), jnp.float32)
    acc = _msg_kernel(n_pad, e, ch)(edge_index, q_rows, z2)

    acc_t = jnp.transpose(acc, (0, 2, 1)).reshape(NC, 2, r, 128)
    o = pl.pallas_call(
        _glue_b,
        out_shape=jax.ShapeDtypeStruct((3, r, 128), jnp.float32),
    )(acc_t, qf, dinv, W, b.reshape(1, 3))
    return o.reshape(3, n_pad)[:, :n].T


# speedup vs baseline: 159.8485x; 159.8485x over previous
import functools

import jax
import jax.numpy as jnp
from jax import lax
from jax.experimental import pallas as pl
from jax.experimental.pallas import tpu as pltpu
from jax.experimental.pallas import tpu_sc as plsc

NC = 2
NS = 16
NW = NC * NS


def _sc_mesh():
    return plsc.VectorSubcoreMesh(core_axis_name="c", subcore_axis_name="s")


def _hist_kernel(n_pad: int, e: int, ch: int):
    per_tile = e // NW
    nch = per_tile // ch
    slc = n_pad // NS

    @functools.partial(
        pl.kernel,
        out_type=jax.ShapeDtypeStruct((NC * n_pad,), jnp.float32),
        mesh=_sc_mesh(),
        scratch_types=[
            pltpu.VMEM((ch,), jnp.int32),
            pltpu.VMEM((ch,), jnp.float32),
            pltpu.VMEM((n_pad // NS,), jnp.float32),
            pltpu.VMEM_SHARED((n_pad,), jnp.float32),
        ],
    )
    def hist(dst_hbm, out_hbm, idx_v, ones_v, stg_v, deg_sh):
        c = lax.axis_index("c")
        s = lax.axis_index("s")

        @pl.loop(0, ch, step=16)
        def _(i):
            ones_v[pl.ds(i, 16)] = jnp.full((16,), 1.0, jnp.float32)

        @pl.loop(0, slc, step=16)
        def _(i):
            stg_v[pl.ds(i, 16)] = jnp.full((16,), 0.0, jnp.float32)

        pltpu.sync_copy(stg_v, deg_sh.at[pl.ds(s * slc, slc)])
        plsc.subcore_barrier()

        base = (c * NS + s) * per_tile

        @pl.loop(0, nch)
        def _(k):
            pltpu.sync_copy(dst_hbm.at[pl.ds(base + k * ch, ch)], idx_v)
            pltpu.sync_copy(ones_v, deg_sh.at[idx_v], add=True)

        plsc.subcore_barrier()
        pltpu.sync_copy(deg_sh.at[pl.ds(s * slc, slc)], stg_v)
        pltpu.sync_copy(stg_v, out_hbm.at[pl.ds(c * n_pad + s * slc, slc)])

    return hist


def _msg_kernel(n_pad: int, e: int, ch: int):
    per_tile = e // NW
    nch = per_tile // ch
    slc = n_pad // NS

    @functools.partial(
        pl.kernel,
        out_type=jax.ShapeDtypeStruct((NC * 2 * n_pad,), jnp.float32),
        mesh=_sc_mesh(),
        scratch_types=[
            pltpu.VMEM((ch,), jnp.int32),
            pltpu.VMEM((ch,), jnp.int32),
            pltpu.VMEM((ch,), jnp.float32),
            pltpu.VMEM((ch,), jnp.float32),
            pltpu.VMEM((n_pad // NS,), jnp.float32),
            pltpu.VMEM_SHARED((n_pad,), jnp.float32),
            pltpu.VMEM_SHARED((n_pad,), jnp.float32),
        ],
    )
    def msg(src_hbm, dst_hbm, q0_hbm, q1_hbm, out_hbm,
            src_v, dst_v, m0_v, m1_v, stg_v, acc0_sh, acc1_sh):
        c = lax.axis_index("c")
        s = lax.axis_index("s")

        @pl.loop(0, slc, step=16)
        def _(i):
            stg_v[pl.ds(i, 16)] = jnp.full((16,), 0.0, jnp.float32)

        pltpu.sync_copy(stg_v, acc0_sh.at[pl.ds(s * slc, slc)])
        pltpu.sync_copy(stg_v, acc1_sh.at[pl.ds(s * slc, slc)])
        plsc.subcore_barrier()

        base = (c * NS + s) * per_tile

        @pl.loop(0, nch)
        def _(k):
            pltpu.sync_copy(src_hbm.at[pl.ds(base + k * ch, ch)], src_v)
            pltpu.sync_copy(dst_hbm.at[pl.ds(base + k * ch, ch)], dst_v)
            pltpu.sync_copy(q0_hbm.at[src_v], m0_v)
            pltpu.sync_copy(q1_hbm.at[src_v], m1_v)
            pltpu.sync_copy(m0_v, acc0_sh.at[dst_v], add=True)
            pltpu.sync_copy(m1_v, acc1_sh.at[dst_v], add=True)

        plsc.subcore_barrier()
        pltpu.sync_copy(acc0_sh.at[pl.ds(s * slc, slc)], stg_v)
        pltpu.sync_copy(stg_v,
                        out_hbm.at[pl.ds((c * 2) * n_pad + s * slc, slc)])
        pltpu.sync_copy(acc1_sh.at[pl.ds(s * slc, slc)], stg_v)
        pltpu.sync_copy(stg_v,
                        out_hbm.at[pl.ds((c * 2 + 1) * n_pad + s * slc, slc)])

    return msg


def _glue_a(x2_ref, p_ref, dinv_ref, q_ref):
    deg = p_ref[0] + p_ref[1] + 1.0
    dinv = lax.rsqrt(deg)
    dinv_ref[...] = dinv
    q_ref[0] = x2_ref[0] * dinv
    q_ref[1] = x2_ref[1] * dinv


def _glue_b(acc_ref, q_ref, dinv_ref, w_ref, b_ref, o_ref):
    s0 = acc_ref[0, 0] + acc_ref[1, 0] + q_ref[0]
    s1 = acc_ref[0, 1] + acc_ref[1, 1] + q_ref[1]
    dinv = dinv_ref[...]
    for k in range(3):
        z = dinv * (s0 * w_ref[0, k] + s1 * w_ref[1, k]) + b_ref[0, k]
        o_ref[k] = jax.nn.sigmoid(z)


def kernel(x, edge_index, W, b):
    n = x.shape[0]
    e = edge_index.shape[1]
    n_pad = -(-n // 128) * 128
    while n_pad % (NS * 8) != 0:
        n_pad += 128
    r = n_pad // 128
    ch = 8000
    assert e % (NW * ch) == 0

    src = edge_index[0]
    dst = edge_index[1]
    p = _hist_kernel(n_pad, e, ch)(dst)

    xt = jnp.zeros((2, n_pad), jnp.float32).at[:, :n].set(x.T)
    x2 = xt.reshape(2, r, 128)
    p2 = p.reshape(NC, r, 128)
    dinv, qf = pl.pallas_call(
        _glue_a,
        out_shape=[
            jax.ShapeDtypeStruct((r, 128), jnp.float32),
            jax.ShapeDtypeStruct((2, r, 128), jnp.float32),
        ],
    )(x2, p2)

    q0 = qf[0].reshape(n_pad)
    q1 = qf[1].reshape(n_pad)
    acc = _msg_kernel(n_pad, e, ch)(src, dst, q0, q1)

    acc_t = acc.reshape(NC, 2, r, 128)
    o = pl.pallas_call(
        _glue_b,
        out_shape=jax.ShapeDtypeStruct((3, r, 128), jnp.float32),
    )(acc_t, qf, dinv, W, b.reshape(1, 3))
    return o.reshape(3, n_pad)[:, :n].T


# speedup vs baseline: 176.1164x; 1.1018x over previous
import functools

import jax
import jax.numpy as jnp
from jax import lax
from jax.experimental import pallas as pl
from jax.experimental.pallas import tpu as pltpu
from jax.experimental.pallas import tpu_sc as plsc

NC = 2
NS = 16
NW = NC * NS


def _sc_mesh():
    return plsc.VectorSubcoreMesh(core_axis_name="c", subcore_axis_name="s")


NSLOT = 4


def _hist_kernel(n_pad: int, e: int, ch: int):
    per_tile = e // NW
    nch = per_tile // ch
    niter = nch // NSLOT
    slc = n_pad // NS

    @functools.partial(
        pl.kernel,
        out_type=jax.ShapeDtypeStruct((NC * n_pad,), jnp.float32),
        mesh=_sc_mesh(),
        scratch_types=(
            [pltpu.VMEM((ch,), jnp.int32) for _ in range(NSLOT)]
            + [pltpu.VMEM((ch,), jnp.float32),
               pltpu.VMEM((n_pad // NS,), jnp.float32),
               pltpu.VMEM_SHARED((n_pad,), jnp.float32)]
            + [pltpu.SemaphoreType.DMA for _ in range(2 * NSLOT)]
        ),
    )
    def hist(dst_hbm, out_hbm, *refs):
        idx_v = refs[:NSLOT]
        ones_v = refs[NSLOT]
        stg_v = refs[NSLOT + 1]
        deg_sh = refs[NSLOT + 2]
        sem_i = refs[NSLOT + 3: 2 * NSLOT + 3]
        sem_s = refs[2 * NSLOT + 3: 3 * NSLOT + 3]
        c = lax.axis_index("c")
        s = lax.axis_index("s")

        @pl.loop(0, ch, step=16)
        def _(i):
            ones_v[pl.ds(i, 16)] = jnp.full((16,), 1.0, jnp.float32)

        @pl.loop(0, slc, step=16)
        def _(i):
            stg_v[pl.ds(i, 16)] = jnp.full((16,), 0.0, jnp.float32)

        pltpu.sync_copy(stg_v, deg_sh.at[pl.ds(s * slc, slc)])
        plsc.subcore_barrier()

        base = (c * NS + s) * per_tile

        @pl.loop(0, niter)
        def _(j):
            kb = base + j * (NSLOT * ch)
            cp_i = [pltpu.async_copy(dst_hbm.at[pl.ds(kb + b * ch, ch)],
                                     idx_v[b], sem_i[b])
                    for b in range(NSLOT)]
            cp_s = []
            for b in range(NSLOT):
                cp_i[b].wait()
                cp_s.append(pltpu.async_copy(ones_v, deg_sh.at[idx_v[b]],
                                             sem_s[b], add=True))
            for cp in cp_s:
                cp.wait()

        plsc.subcore_barrier()
        pltpu.sync_copy(deg_sh.at[pl.ds(s * slc, slc)], stg_v)
        pltpu.sync_copy(stg_v, out_hbm.at[pl.ds(c * n_pad + s * slc, slc)])

    return hist


def _msg_kernel(n_pad: int, e: int, ch: int):
    per_tile = e // NW
    nch = per_tile // ch
    niter = nch // NSLOT
    slc = n_pad // NS

    @functools.partial(
        pl.kernel,
        out_type=jax.ShapeDtypeStruct((NC * 2 * n_pad,), jnp.float32),
        mesh=_sc_mesh(),
        scratch_types=(
            [pltpu.VMEM((ch,), jnp.int32) for _ in range(2 * NSLOT)]
            + [pltpu.VMEM((ch,), jnp.float32) for _ in range(2 * NSLOT)]
            + [pltpu.VMEM((n_pad // NS,), jnp.float32),
               pltpu.VMEM_SHARED((n_pad,), jnp.float32),
               pltpu.VMEM_SHARED((n_pad,), jnp.float32)]
            + [pltpu.SemaphoreType.DMA for _ in range(3 * NSLOT)]
        ),
    )
    def msg(src_hbm, dst_hbm, q0_hbm, q1_hbm, out_hbm, *refs):
        src_v = refs[0:NSLOT]
        dst_v = refs[NSLOT:2 * NSLOT]
        m0_v = refs[2 * NSLOT:3 * NSLOT]
        m1_v = refs[3 * NSLOT:4 * NSLOT]
        stg_v = refs[4 * NSLOT]
        acc0_sh = refs[4 * NSLOT + 1]
        acc1_sh = refs[4 * NSLOT + 2]
        sem_i = refs[4 * NSLOT + 3: 5 * NSLOT + 3]
        sem_g = refs[5 * NSLOT + 3: 6 * NSLOT + 3]
        sem_s = refs[6 * NSLOT + 3: 7 * NSLOT + 3]
        c = lax.axis_index("c")
        s = lax.axis_index("s")

        @pl.loop(0, slc, step=16)
        def _(i):
            stg_v[pl.ds(i, 16)] = jnp.full((16,), 0.0, jnp.float32)

        pltpu.sync_copy(stg_v, acc0_sh.at[pl.ds(s * slc, slc)])
        pltpu.sync_copy(stg_v, acc1_sh.at[pl.ds(s * slc, slc)])
        plsc.subcore_barrier()

        base = (c * NS + s) * per_tile

        @pl.loop(0, niter)
        def _(j):
            kb = base + j * (NSLOT * ch)
            cp_i = []
            for b in range(NSLOT):
                cp_i.append((
                    pltpu.async_copy(src_hbm.at[pl.ds(kb + b * ch, ch)],
                                     src_v[b], sem_i[b]),
                    pltpu.async_copy(dst_hbm.at[pl.ds(kb + b * ch, ch)],
                                     dst_v[b], sem_i[b]),
                ))
            cp_g = []
            for b in range(NSLOT):
                cp_i[b][0].wait()
                cp_i[b][1].wait()
                cp_g.append((
                    pltpu.async_copy(q0_hbm.at[src_v[b]], m0_v[b], sem_g[b]),
                    pltpu.async_copy(q1_hbm.at[src_v[b]], m1_v[b], sem_g[b]),
                ))
            cp_s = []
            for b in range(NSLOT):
                cp_g[b][0].wait()
                cp_g[b][1].wait()
                cp_s.append((
                    pltpu.async_copy(m0_v[b], acc0_sh.at[dst_v[b]],
                                     sem_s[b], add=True),
                    pltpu.async_copy(m1_v[b], acc1_sh.at[dst_v[b]],
                                     sem_s[b], add=True),
                ))
            for b in range(NSLOT):
                cp_s[b][0].wait()
                cp_s[b][1].wait()

        plsc.subcore_barrier()
        pltpu.sync_copy(acc0_sh.at[pl.ds(s * slc, slc)], stg_v)
        pltpu.sync_copy(stg_v,
                        out_hbm.at[pl.ds((c * 2) * n_pad + s * slc, slc)])
        pltpu.sync_copy(acc1_sh.at[pl.ds(s * slc, slc)], stg_v)
        pltpu.sync_copy(stg_v,
                        out_hbm.at[pl.ds((c * 2 + 1) * n_pad + s * slc, slc)])

    return msg


def _glue_a(x2_ref, p_ref, dinv_ref, q_ref):
    deg = p_ref[0] + p_ref[1] + 1.0
    dinv = lax.rsqrt(deg)
    dinv_ref[...] = dinv
    q_ref[0] = x2_ref[0] * dinv
    q_ref[1] = x2_ref[1] * dinv


def _glue_b(acc_ref, q_ref, dinv_ref, w_ref, b_ref, o_ref):
    s0 = acc_ref[0, 0] + acc_ref[1, 0] + q_ref[0]
    s1 = acc_ref[0, 1] + acc_ref[1, 1] + q_ref[1]
    dinv = dinv_ref[...]
    for k in range(3):
        z = dinv * (s0 * w_ref[0, k] + s1 * w_ref[1, k]) + b_ref[0, k]
        o_ref[k] = jax.nn.sigmoid(z)


def kernel(x, edge_index, W, b):
    n = x.shape[0]
    e = edge_index.shape[1]
    n_pad = -(-n // 128) * 128
    while n_pad % (NS * 8) != 0:
        n_pad += 128
    r = n_pad // 128
    ch = 5000
    assert e % (NW * ch * NSLOT) == 0

    src = edge_index[0]
    dst = edge_index[1]
    p = _hist_kernel(n_pad, e, ch)(dst)

    xt = jnp.zeros((2, n_pad), jnp.float32).at[:, :n].set(x.T)
    x2 = xt.reshape(2, r, 128)
    p2 = p.reshape(NC, r, 128)
    dinv, qf = pl.pallas_call(
        _glue_a,
        out_shape=[
            jax.ShapeDtypeStruct((r, 128), jnp.float32),
            jax.ShapeDtypeStruct((2, r, 128), jnp.float32),
        ],
    )(x2, p2)

    q0 = qf[0].reshape(n_pad)
    q1 = qf[1].reshape(n_pad)
    acc = _msg_kernel(n_pad, e, ch)(src, dst, q0, q1)

    acc_t = acc.reshape(NC, 2, r, 128)
    o = pl.pallas_call(
        _glue_b,
        out_shape=jax.ShapeDtypeStruct((3, r, 128), jnp.float32),
    )(acc_t, qf, dinv, W, b.reshape(1, 3))
    return o.reshape(3, n_pad)[:, :n].T


# speedup vs baseline: 320.0120x; 1.8170x over previous
import functools

import jax
import jax.numpy as jnp
from jax import lax
from jax.experimental import pallas as pl
from jax.experimental.pallas import tpu as pltpu
from jax.experimental.pallas import tpu_sc as plsc

NC = 2
NS = 16
NW = NC * NS


def _sc_mesh():
    return plsc.VectorSubcoreMesh(core_axis_name="c", subcore_axis_name="s")


NSLOT = 4


def _hist_kernel(n_pad: int, e: int, ch: int):
    per_tile = e // NW
    nch = per_tile // ch
    niter = nch // NSLOT
    slc = n_pad // NS

    @functools.partial(
        pl.kernel,
        out_type=jax.ShapeDtypeStruct((NC * n_pad,), jnp.float32),
        mesh=_sc_mesh(),
        scratch_types=(
            [pltpu.VMEM((ch,), jnp.int32) for _ in range(NSLOT)]
            + [pltpu.VMEM((ch,), jnp.float32),
               pltpu.VMEM((n_pad // NS,), jnp.float32),
               pltpu.VMEM_SHARED((n_pad,), jnp.float32)]
            + [pltpu.SemaphoreType.DMA for _ in range(2 * NSLOT)]
        ),
    )
    def hist(dst_hbm, out_hbm, *refs):
        idx_v = refs[:NSLOT]
        ones_v = refs[NSLOT]
        stg_v = refs[NSLOT + 1]
        deg_sh = refs[NSLOT + 2]
        sem_i = refs[NSLOT + 3: 2 * NSLOT + 3]
        sem_s = refs[2 * NSLOT + 3: 3 * NSLOT + 3]
        c = lax.axis_index("c")
        s = lax.axis_index("s")

        @pl.loop(0, ch, step=16)
        def _(i):
            ones_v[pl.ds(i, 16)] = jnp.full((16,), 1.0, jnp.float32)

        @pl.loop(0, slc, step=16)
        def _(i):
            stg_v[pl.ds(i, 16)] = jnp.full((16,), 0.0, jnp.float32)

        pltpu.sync_copy(stg_v, deg_sh.at[pl.ds(s * slc, slc)])
        plsc.subcore_barrier()

        base = (c * NS + s) * per_tile

        @pl.loop(0, niter)
        def _(j):
            kb = base + j * (NSLOT * ch)
            cp_i = [pltpu.async_copy(dst_hbm.at[pl.ds(kb + b * ch, ch)],
                                     idx_v[b], sem_i[b])
                    for b in range(NSLOT)]
            cp_s = []
            for b in range(NSLOT):
                cp_i[b].wait()
                cp_s.append(pltpu.async_copy(ones_v, deg_sh.at[idx_v[b]],
                                             sem_s[b], add=True))
            for cp in cp_s:
                cp.wait()

        plsc.subcore_barrier()
        pltpu.sync_copy(deg_sh.at[pl.ds(s * slc, slc)], stg_v)
        pltpu.sync_copy(stg_v, out_hbm.at[pl.ds(c * n_pad + s * slc, slc)])

    return hist


def _msg_kernel(n_pad: int, e: int, ch: int):
    per_tile = e // NW
    nch = per_tile // ch
    niter = nch // NSLOT
    slc = n_pad // NS

    @functools.partial(
        pl.kernel,
        out_type=jax.ShapeDtypeStruct((NC * 2 * n_pad,), jnp.float32),
        mesh=_sc_mesh(),
        scratch_types=(
            [pltpu.VMEM((ch,), jnp.int32) for _ in range(2 * NSLOT)]
            + [pltpu.VMEM((ch,), jnp.float32) for _ in range(2 * NSLOT)]
            + [pltpu.VMEM((n_pad // NS,), jnp.float32),
               pltpu.VMEM_SHARED((n_pad,), jnp.float32),
               pltpu.VMEM_SHARED((n_pad,), jnp.float32),
               pltpu.VMEM_SHARED((n_pad,), jnp.float32),
               pltpu.VMEM_SHARED((n_pad,), jnp.float32)]
            + [pltpu.SemaphoreType.DMA for _ in range(3 * NSLOT)]
        ),
    )
    def msg(src_hbm, dst_hbm, q0_hbm, q1_hbm, out_hbm, *refs):
        src_v = refs[0:NSLOT]
        dst_v = refs[NSLOT:2 * NSLOT]
        m0_v = refs[2 * NSLOT:3 * NSLOT]
        m1_v = refs[3 * NSLOT:4 * NSLOT]
        stg_v = refs[4 * NSLOT]
        acc0_sh = refs[4 * NSLOT + 1]
        acc1_sh = refs[4 * NSLOT + 2]
        q0_sh = refs[4 * NSLOT + 3]
        q1_sh = refs[4 * NSLOT + 4]
        sem_i = refs[4 * NSLOT + 5: 5 * NSLOT + 5]
        sem_g = refs[5 * NSLOT + 5: 6 * NSLOT + 5]
        sem_s = refs[6 * NSLOT + 5: 7 * NSLOT + 5]
        c = lax.axis_index("c")
        s = lax.axis_index("s")

        pltpu.sync_copy(q0_hbm.at[pl.ds(s * slc, slc)], stg_v)
        pltpu.sync_copy(stg_v, q0_sh.at[pl.ds(s * slc, slc)])
        pltpu.sync_copy(q1_hbm.at[pl.ds(s * slc, slc)], stg_v)
        pltpu.sync_copy(stg_v, q1_sh.at[pl.ds(s * slc, slc)])

        @pl.loop(0, slc, step=16)
        def _(i):
            stg_v[pl.ds(i, 16)] = jnp.full((16,), 0.0, jnp.float32)

        pltpu.sync_copy(stg_v, acc0_sh.at[pl.ds(s * slc, slc)])
        pltpu.sync_copy(stg_v, acc1_sh.at[pl.ds(s * slc, slc)])
        plsc.subcore_barrier()

        base = (c * NS + s) * per_tile

        @pl.loop(0, niter)
        def _(j):
            kb = base + j * (NSLOT * ch)
            cp_i = []
            for b in range(NSLOT):
                cp_i.append((
                    pltpu.async_copy(src_hbm.at[pl.ds(kb + b * ch, ch)],
                                     src_v[b], sem_i[b]),
                    pltpu.async_copy(dst_hbm.at[pl.ds(kb + b * ch, ch)],
                                     dst_v[b], sem_i[b]),
                ))
            cp_g = []
            for b in range(NSLOT):
                cp_i[b][0].wait()
                cp_i[b][1].wait()
                cp_g.append((
                    pltpu.async_copy(q0_sh.at[src_v[b]], m0_v[b], sem_g[b]),
                    pltpu.async_copy(q1_sh.at[src_v[b]], m1_v[b], sem_g[b]),
                ))
            cp_s = []
            for b in range(NSLOT):
                cp_g[b][0].wait()
                cp_g[b][1].wait()
                cp_s.append((
                    pltpu.async_copy(m0_v[b], acc0_sh.at[dst_v[b]],
                                     sem_s[b], add=True),
                    pltpu.async_copy(m1_v[b], acc1_sh.at[dst_v[b]],
                                     sem_s[b], add=True),
                ))
            for b in range(NSLOT):
                cp_s[b][0].wait()
                cp_s[b][1].wait()

        plsc.subcore_barrier()
        pltpu.sync_copy(acc0_sh.at[pl.ds(s * slc, slc)], stg_v)
        pltpu.sync_copy(stg_v,
                        out_hbm.at[pl.ds((c * 2) * n_pad + s * slc, slc)])
        pltpu.sync_copy(acc1_sh.at[pl.ds(s * slc, slc)], stg_v)
        pltpu.sync_copy(stg_v,
                        out_hbm.at[pl.ds((c * 2 + 1) * n_pad + s * slc, slc)])

    return msg


def _glue_a(x2_ref, p_ref, dinv_ref, q_ref):
    deg = p_ref[0] + p_ref[1] + 1.0
    dinv = lax.rsqrt(deg)
    dinv_ref[...] = dinv
    q_ref[0] = x2_ref[0] * dinv
    q_ref[1] = x2_ref[1] * dinv


def _glue_b(acc_ref, q_ref, dinv_ref, w_ref, b_ref, o_ref):
    s0 = acc_ref[0, 0] + acc_ref[1, 0] + q_ref[0]
    s1 = acc_ref[0, 1] + acc_ref[1, 1] + q_ref[1]
    dinv = dinv_ref[...]
    for k in range(3):
        z = dinv * (s0 * w_ref[0, k] + s1 * w_ref[1, k]) + b_ref[0, k]
        o_ref[k] = jax.nn.sigmoid(z)


def kernel(x, edge_index, W, b):
    n = x.shape[0]
    e = edge_index.shape[1]
    n_pad = -(-n // 128) * 128
    while n_pad % (NS * 8) != 0:
        n_pad += 128
    r = n_pad // 128
    ch = 5000
    assert e % (NW * ch * NSLOT) == 0

    src = edge_index[0]
    dst = edge_index[1]
    p = _hist_kernel(n_pad, e, ch)(dst)

    xt = jnp.zeros((2, n_pad), jnp.float32).at[:, :n].set(x.T)
    x2 = xt.reshape(2, r, 128)
    p2 = p.reshape(NC, r, 128)
    dinv, qf = pl.pallas_call(
        _glue_a,
        out_shape=[
            jax.ShapeDtypeStruct((r, 128), jnp.float32),
            jax.ShapeDtypeStruct((2, r, 128), jnp.float32),
        ],
    )(x2, p2)

    q0 = qf[0].reshape(n_pad)
    q1 = qf[1].reshape(n_pad)
    acc = _msg_kernel(n_pad, e, ch)(src, dst, q0, q1)

    acc_t = acc.reshape(NC, 2, r, 128)
    o = pl.pallas_call(
        _glue_b,
        out_shape=jax.ShapeDtypeStruct((3, r, 128), jnp.float32),
    )(acc_t, qf, dinv, W, b.reshape(1, 3))
    return o.reshape(3, n_pad)[:, :n].T
